# baseline (speedup 1.0000x reference)
import functools

import jax
import jax.numpy as jnp
from jax import lax
from jax.experimental import pallas as pl
from jax.experimental.pallas import tpu as pltpu
from jax.experimental.pallas import tpu_sc as plsc

_B = 16
_Q = 4096
_L = 16
_NCHUNK = 1
_QC = _Q // _NCHUNK
_NITER = _QC // _L

_LOG1P_COEFS = (
    0.000502721633151848,
    0.9823971197982744,
    -0.3971182964499665,
    0.10774685617806044,
)


def _ce_body(pred_hbm, tgt_hbm, objn_hbm, out_hbm,
             row_v, tgt_v, objn_v, stage_v, allrows_v, shared,
             sem_o, *sems):
    sem_p = sems[:_NCHUNK]
    sem_t = sems[_NCHUNK:]
    s = lax.axis_index("s")
    nrow = 2 * _Q // 128
    nrc = nrow // _NCHUNK
    cp_o = pltpu.async_copy(objn_hbm, objn_v, sem_o)
    cps = []
    for k in range(_NCHUNK):
        cps.append((
            pltpu.async_copy(
                pred_hbm.at[pl.ds(s * nrow + k * nrc, nrc)],
                row_v.at[pl.ds(k * nrc, nrc)], sem_p[k]),
            pltpu.async_copy(
                tgt_hbm.at[s, pl.ds(k * _QC, _QC)],
                tgt_v.at[pl.ds(k * _QC, _QC)], sem_t[k]),
        ))

    cp_o.wait()
    iota = lax.broadcasted_iota(jnp.int32, (_L,), 0)
    sful = jnp.full((_L,), s, jnp.int32)
    my_numb = plsc.load_gather(objn_v, [sful])
    inv = (1.0 / _B) / my_numb.astype(jnp.float32)

    def step(i, acc):
        r = (i >> 3) * 2
        j0 = (i & 7) * _L
        g0 = row_v[r, pl.ds(j0, _L)]
        g1 = row_v[r + 1, pl.ds(j0, _L)]
        t = tgt_v[pl.ds(i * _L, _L)]
        m = jnp.maximum(g0, g1)
        e = jnp.exp(jnp.minimum(g0, g1) - m)
        lp = jnp.full((_L,), _LOG1P_COEFS[-1], jnp.float32)
        for q in _LOG1P_COEFS[-2::-1]:
            lp = lp * e + q
        pt = jnp.where(t == 0, g0, g1)
        ce = (m - pt) + lp
        return acc + jnp.where(iota + i * _L < my_numb, ce, 0.0)

    acc = jnp.zeros((_L,), jnp.float32)
    for k in range(_NCHUNK):
        cps[k][0].wait()
        cps[k][1].wait()
        acc = plsc.parallel_loop(
            k * _NITER, (k + 1) * _NITER, 1, unroll=4, carry=acc)(step)

    stage_v[...] = acc * inv
    pltpu.sync_copy(stage_v, shared.at[pl.ds(s * _L, _L)])
    plsc.subcore_barrier()

    @pl.when(s == 0)
    def _():
        pltpu.sync_copy(shared, allrows_v)

        def red(ss, tot):
            return tot + allrows_v[pl.ds(ss * _L, _L)]

        tot = lax.fori_loop(0, _B, red, jnp.zeros((_L,), jnp.float32))
        stage_v[...] = jnp.full((_L,), jnp.sum(tot), jnp.float32)
        pltpu.sync_copy(stage_v, out_hbm)


_sc_celoss = functools.partial(
    pl.kernel,
    out_type=jax.ShapeDtypeStruct((_L,), jnp.float32),
    mesh=plsc.VectorSubcoreMesh(
        core_axis_name="c", subcore_axis_name="s", num_cores=1),
    compiler_params=pltpu.CompilerParams(
        needs_layout_passes=False, use_tc_tiling_on_sc=True),
    scratch_types=[
        pltpu.VMEM((2 * _Q // 128, 128), jnp.float32),
        pltpu.VMEM((_Q,), jnp.int32),
        pltpu.VMEM((_B,), jnp.int32),
        pltpu.VMEM((_L,), jnp.float32),
        pltpu.VMEM((_B * _L,), jnp.float32),
        pltpu.VMEM_SHARED((_B * _L,), jnp.float32),
    ] + [pltpu.SemaphoreType.DMA] * (1 + 2 * _NCHUNK),
)(_ce_body)


def kernel(pred, target, object_num):
    pred2 = pred.reshape(_B, _Q // 128, 128, 2).transpose(0, 1, 3, 2)
    pred2 = pred2.reshape(_B * (_Q // 128) * 2, 128)
    out = _sc_celoss(pred2, target.astype(jnp.int32),
                     object_num.astype(jnp.int32))
    return out[0]
